# Initial kernel scaffold; baseline (speedup 1.0000x reference)
import jax
import jax.numpy as jnp
from jax.experimental import pallas as pl


def kernel(Xc, tables):
    raise NotImplementedError("write your pallas kernel here")


# speedup vs baseline: 1.8218x; 1.8218x over previous
import functools

import jax
import jax.numpy as jnp
from jax import lax
from jax.experimental import pallas as pl
from jax.experimental.pallas import tpu as pltpu
from jax.experimental.pallas import tpu_sc as plsc

NUM_FIELDS = 26
EMB_DIM = 32
NC = 2
NS = 16
NW = NC * NS

CHUNK = 1600


def _make_gather(n_rows: int):
    per_w = n_rows // NW
    n_chunks = per_w // CHUNK
    mesh = plsc.VectorSubcoreMesh(
        core_axis_name="c", subcore_axis_name="s", num_cores=NC, num_subcores=NS
    )

    @functools.partial(
        pl.kernel,
        out_type=jax.ShapeDtypeStruct((n_rows, EMB_DIM), jnp.float32),
        mesh=mesh,
        scratch_types=[
            pltpu.VMEM((CHUNK,), jnp.int32),
            pltpu.VMEM((CHUNK, EMB_DIM), jnp.float32),
            pltpu.SemaphoreType.DMA,
        ],
        compiler_params=pltpu.CompilerParams(use_tc_tiling_on_sc=False),
    )
    def gather_kernel(table_hbm, idx_hbm, out_hbm, idx_v, rows_v, sem):
        wid = lax.axis_index("s") * NC + lax.axis_index("c")
        base = wid * per_w

        def body(i, carry):
            off = base + i * CHUNK
            pltpu.sync_copy(idx_hbm.at[pl.ds(off, CHUNK)], idx_v)
            pltpu.async_copy(table_hbm.at[idx_v], rows_v, sem).wait()
            pltpu.sync_copy(rows_v, out_hbm.at[pl.ds(off, CHUNK)])
            return carry

        lax.fori_loop(0, n_chunks, body, 0)

    return gather_kernel


def kernel(Xc, tables):
    B, H, F = Xc.shape
    V = tables.shape[1]
    table_flat = tables.reshape(F * V, EMB_DIM)
    offsets = (jnp.arange(F, dtype=jnp.int32) * V)[None, None, :]
    idx = (Xc.astype(jnp.int32) + offsets).reshape(-1)
    out_flat = _make_gather(idx.shape[0])(table_flat, idx)
    return out_flat.reshape(B, H, F * EMB_DIM)


# speedup vs baseline: 1.8450x; 1.0127x over previous
import functools

import jax
import jax.numpy as jnp
from jax import lax
from jax.experimental import pallas as pl
from jax.experimental.pallas import tpu as pltpu
from jax.experimental.pallas import tpu_sc as plsc

NUM_FIELDS = 26
EMB_DIM = 32
NC = 2
NS = 16
NW = NC * NS

CHUNK = 1040


def _make_gather(n_rows: int):
    per_w = n_rows // NW
    n_chunks = per_w // CHUNK
    assert per_w % CHUNK == 0 and n_chunks % 2 == 0
    n2 = n_chunks // 2
    mesh = plsc.VectorSubcoreMesh(
        core_axis_name="c", subcore_axis_name="s", num_cores=NC, num_subcores=NS
    )

    @functools.partial(
        pl.kernel,
        out_type=jax.ShapeDtypeStruct((n_rows, EMB_DIM), jnp.float32),
        mesh=mesh,
        scratch_types=[
            pltpu.VMEM((per_w,), jnp.int32),
            pltpu.VMEM((CHUNK, EMB_DIM), jnp.float32),
            pltpu.VMEM((CHUNK, EMB_DIM), jnp.float32),
            pltpu.SemaphoreType.DMA,
            pltpu.SemaphoreType.DMA,
            pltpu.SemaphoreType.DMA,
            pltpu.SemaphoreType.DMA,
        ],
        compiler_params=pltpu.CompilerParams(use_tc_tiling_on_sc=False),
    )
    def gather_kernel(table_hbm, idx_hbm, out_hbm,
                      idx_v, rows0, rows1, gsem0, gsem1, wsem0, wsem1):
        wid = lax.axis_index("s") * NC + lax.axis_index("c")
        base = wid * per_w

        pltpu.sync_copy(idx_hbm.at[pl.ds(base, per_w)], idx_v)

        def gather(i, buf, sem):
            pltpu.async_copy(
                table_hbm.at[idx_v.at[pl.ds(i * CHUNK, CHUNK)]], buf, sem)

        def gather_wait(buf, sem):
            pltpu.make_async_copy(
                table_hbm.at[idx_v.at[pl.ds(0, CHUNK)]], buf, sem).wait()

        def writeback(i, buf, sem):
            pltpu.async_copy(
                buf, out_hbm.at[pl.ds(base + i * CHUNK, CHUNK)], sem)

        def writeback_wait(buf, sem):
            pltpu.make_async_copy(
                buf, out_hbm.at[pl.ds(base, CHUNK)], sem).wait()

        gather(0, rows0, gsem0)

        def body(j, carry):
            i0 = 2 * j
            @pl.when(j > 0)
            def _():
                writeback_wait(rows1, wsem1)

            gather(i0 + 1, rows1, gsem1)
            gather_wait(rows0, gsem0)
            writeback(i0, rows0, wsem0)
            gather_wait(rows1, gsem1)
            writeback(i0 + 1, rows1, wsem1)

            @pl.when(j < n2 - 1)
            def _():
                writeback_wait(rows0, wsem0)
                gather(i0 + 2, rows0, gsem0)

            return carry

        lax.fori_loop(0, n2, body, 0)
        writeback_wait(rows0, wsem0)
        writeback_wait(rows1, wsem1)

    return gather_kernel


def kernel(Xc, tables):
    B, H, F = Xc.shape
    V = tables.shape[1]
    table_flat = tables.reshape(F * V, EMB_DIM)
    offsets = (jnp.arange(F, dtype=jnp.int32) * V)[None, None, :]
    idx = (Xc.astype(jnp.int32) + offsets).reshape(-1)
    out_flat = _make_gather(idx.shape[0])(table_flat, idx)
    return out_flat.reshape(B, H, F * EMB_DIM)


# speedup vs baseline: 1.8578x; 1.0069x over previous
import functools

import jax
import jax.numpy as jnp
from jax import lax
from jax.experimental import pallas as pl
from jax.experimental.pallas import tpu as pltpu
from jax.experimental.pallas import tpu_sc as plsc

NUM_FIELDS = 26
EMB_DIM = 32
NC = 2
NS = 16
NW = NC * NS


def _make_gather(B, H, F, V):
    R = B * H
    per_w = R // NW
    mesh = plsc.VectorSubcoreMesh(
        core_axis_name="c", subcore_axis_name="s", num_cores=NC, num_subcores=NS
    )

    @functools.partial(
        pl.kernel,
        out_type=jax.ShapeDtypeStruct((R, F * EMB_DIM), jnp.float32),
        mesh=mesh,
        scratch_types=[
            pltpu.VMEM((per_w,), jnp.int32),
            pltpu.VMEM((per_w,), jnp.int32),
            pltpu.VMEM((per_w, EMB_DIM), jnp.float32),
            pltpu.VMEM((per_w, EMB_DIM), jnp.float32),
            pltpu.SemaphoreType.DMA,
            pltpu.SemaphoreType.DMA,
            pltpu.SemaphoreType.DMA,
            pltpu.SemaphoreType.DMA,
        ],
        compiler_params=pltpu.CompilerParams(use_tc_tiling_on_sc=False),
    )
    def gather_kernel(tables_hbm, idxT_hbm, out_hbm,
                      idx0, idx1, rows0, rows1, gsem0, gsem1, wsem0, wsem1):
        wid = lax.axis_index("s") * NC + lax.axis_index("c")
        base = wid * per_w

        def load_idx(f, ibuf):
            pltpu.sync_copy(idxT_hbm.at[f].at[pl.ds(base, per_w)], ibuf)

        def gather(f, ibuf, buf, sem):
            pltpu.async_copy(tables_hbm.at[f].at[ibuf], buf, sem)

        def gather_wait(ibuf, buf, sem):
            pltpu.make_async_copy(tables_hbm.at[0].at[ibuf], buf, sem).wait()

        def writeback(f, buf, sem):
            pltpu.async_copy(
                buf, out_hbm.at[pl.ds(base, per_w), pl.ds(f * EMB_DIM, EMB_DIM)],
                sem)

        def writeback_wait(buf, sem):
            pltpu.make_async_copy(
                buf, out_hbm.at[pl.ds(base, per_w), pl.ds(0, EMB_DIM)],
                sem).wait()

        load_idx(0, idx0)
        gather(0, idx0, rows0, gsem0)

        def body(j, carry):
            f0 = 2 * j
            @pl.when(j > 0)
            def _():
                writeback_wait(rows1, wsem1)

            load_idx(f0 + 1, idx1)
            gather(f0 + 1, idx1, rows1, gsem1)
            gather_wait(idx0, rows0, gsem0)
            writeback(f0, rows0, wsem0)
            gather_wait(idx1, rows1, gsem1)
            writeback(f0 + 1, rows1, wsem1)

            @pl.when(j < F // 2 - 1)
            def _():
                writeback_wait(rows0, wsem0)
                load_idx(f0 + 2, idx0)
                gather(f0 + 2, idx0, rows0, gsem0)

            return carry

        lax.fori_loop(0, F // 2, body, 0)
        writeback_wait(rows0, wsem0)
        writeback_wait(rows1, wsem1)

    return gather_kernel


def kernel(Xc, tables):
    B, H, F = Xc.shape
    V = tables.shape[1]
    idxT = Xc.reshape(B * H, F).T.astype(jnp.int32)
    out2d = _make_gather(B, H, F, V)(tables, idxT)
    return out2d.reshape(B, H, F * EMB_DIM)
